# baseline (device time: 109638 ns/iter reference)
import jax
import jax.numpy as jnp
from jax import lax
from jax.experimental import pallas as pl
from jax.experimental.pallas import tpu as pltpu

N_DEV = 4
N_EXPERTS = 16
E_LOCAL = 4
CAPACITY = 204
D_MODEL = 512
D_HIDDEN = 1024
N_TOK = 1024


def kernel(x, router_W, route_idx, expert_W):
    del router_W

    def body(x_ref, ridx_ref, ew_ref, out_ref,
             hcomm_ref, ecr_ref, ecl_ref,
             hsend, hrecv, esend_r, erecv_r, esend_l, erecv_l):
        me = lax.axis_index("i")
        right = (me + 1) % N_DEV
        left = (me + 3) % N_DEV

        with jax.named_scope("barrier"):
            barrier_sem = pltpu.get_barrier_semaphore()
            for nbr in (left, right):
                pl.semaphore_signal(
                    barrier_sem, inc=1,
                    device_id=(nbr,), device_id_type=pl.DeviceIdType.MESH,
                )
            pl.semaphore_wait(barrier_sem, 2)

        def mk_rdma(h, comm_ref, send_sems, recv_sems, target):
            src_slot, dst_slot = h % 2, (h + 1) % 2
            return pltpu.make_async_remote_copy(
                src_ref=comm_ref.at[src_slot],
                dst_ref=comm_ref.at[dst_slot],
                send_sem=send_sems.at[src_slot],
                recv_sem=recv_sems.at[dst_slot],
                device_id=(target,),
                device_id_type=pl.DeviceIdType.MESH,
            )

        ew_bf = ew_ref[:, :, :].astype(jnp.bfloat16)
        ecr_ref[0] = ew_bf[0:2]
        ecl_ref[0] = ew_bf[2:4]
        rdma_r = mk_rdma(0, ecr_ref, esend_r, erecv_r, right)
        rdma_l = mk_rdma(0, ecl_ref, esend_l, erecv_l, left)
        with jax.named_scope("ering_send#hop=0"):
            rdma_r.start()
            rdma_l.start()

        ridx = ridx_ref[:, :]
        e_iota = lax.broadcasted_iota(jnp.int32, (N_TOK, N_EXPERTS), 1)
        onehot = (ridx == e_iota).astype(jnp.float32)
        hist = jnp.sum(onehot, axis=0).astype(jnp.int32)
        hcomm_ref[0, 0:1, 0:N_EXPERTS] = hist.reshape(1, N_EXPERTS)

        prev = jnp.zeros((N_EXPERTS,), jnp.float32)
        with jax.named_scope("hist_ring"):
            for h in range(N_DEV - 1):
                hrdma = mk_rdma(h, hcomm_ref, hsend, hrecv, right)
                hrdma.start()
                hrdma.wait()
                origin = (me + (N_DEV - 1 - h)) % N_DEV
                w = (origin < me).astype(jnp.float32)
                prev = prev + w * hcomm_ref[(h + 1) % 2, 0, 0:N_EXPERTS].astype(
                    jnp.float32)

        with jax.named_scope("accept_mask"):
            lmat = (lax.broadcasted_iota(jnp.int32, (N_TOK, N_TOK), 0)
                    > lax.broadcasted_iota(jnp.int32, (N_TOK, N_TOK), 1)
                    ).astype(jnp.float32)
            ranks = jnp.dot(lmat, onehot,
                            preferred_element_type=jnp.float32)
            my_rank = jnp.sum(ranks * onehot, axis=1, keepdims=True)
            my_prev = jnp.sum(onehot * prev[None, :], axis=1, keepdims=True)
            accepted = ((my_prev + my_rank) < float(CAPACITY)).astype(
                jnp.float32)

        x_bf = x_ref[:, :].astype(jnp.bfloat16)

        def acc_expert(g, w_bf, first):
            sel = (ridx == g).astype(jnp.bfloat16)
            d = jnp.dot(x_bf * sel, w_bf, preferred_element_type=jnp.float32)
            if first:
                out_ref[:, :] = d
            else:
                out_ref[:, :] += d

        def compute_pair(h, w_r_of, w_l_of):
            owner_r = (me + (N_DEV - h)) % N_DEV
            owner_l = (me + h) % N_DEV
            for j in (0, 1):
                acc_expert(owner_r * E_LOCAL + j, w_r_of(j), h == 0 and j == 0)
            for j in (2, 3):
                acc_expert(owner_l * E_LOCAL + j, w_l_of(j - 2), False)

        for h in range(N_DEV - 1):
            if h > 0:
                rdma_r = mk_rdma(h, ecr_ref, esend_r, erecv_r, right)
                rdma_l = mk_rdma(h, ecl_ref, esend_l, erecv_l, left)
                with jax.named_scope(f"ering_send#hop={h}"):
                    rdma_r.start()
                    rdma_l.start()
            with jax.named_scope(f"ering_compute#hop={h}"):
                if h == 0:
                    compute_pair(0, lambda j: ew_bf[j], lambda j: ew_bf[2 + j])
                else:
                    s = h % 2
                    compute_pair(h,
                                 lambda j, s=s: ecr_ref[s, j],
                                 lambda j, s=s: ecl_ref[s, j])
            with jax.named_scope(f"ering_wait#hop={h}"):
                rdma_r.wait()
                rdma_l.wait()
        with jax.named_scope("ering_compute#hop=3"):
            compute_pair(N_DEV - 1,
                         lambda j: ecr_ref[1, j],
                         lambda j: ecl_ref[1, j])

        with jax.named_scope("apply_accept"):
            out_ref[:, :] *= accepted

    out_shape = jax.ShapeDtypeStruct((N_TOK, D_HIDDEN), jnp.float32)
    return pl.pallas_call(
        body,
        out_shape=out_shape,
        in_specs=[pl.BlockSpec(memory_space=pltpu.VMEM)] * 3,
        out_specs=pl.BlockSpec(memory_space=pltpu.VMEM),
        scratch_shapes=[
            pltpu.VMEM((2, 8, 128), jnp.int32),
            pltpu.VMEM((2, 2, D_MODEL, D_HIDDEN), jnp.bfloat16),
            pltpu.VMEM((2, 2, D_MODEL, D_HIDDEN), jnp.bfloat16),
            pltpu.SemaphoreType.DMA((2,)),
            pltpu.SemaphoreType.DMA((2,)),
            pltpu.SemaphoreType.DMA((2,)),
            pltpu.SemaphoreType.DMA((2,)),
            pltpu.SemaphoreType.DMA((2,)),
            pltpu.SemaphoreType.DMA((2,)),
        ],
        compiler_params=pltpu.CompilerParams(
            collective_id=0, vmem_limit_bytes=100 * 1024 * 1024),
    )(x, route_idx, expert_W)


# device time: 93147 ns/iter; 1.1770x vs baseline; 1.1770x over previous
import jax
import jax.numpy as jnp
from jax import lax
from jax.experimental import pallas as pl
from jax.experimental.pallas import tpu as pltpu

N_DEV = 4
N_EXPERTS = 16
E_LOCAL = 4
CAPACITY = 204
D_MODEL = 512
D_HIDDEN = 1024
N_TOK = 1024


def kernel(x, router_W, route_idx, expert_W):
    del router_W

    def body(x_ref, ridx_ref, ew_ref, out_ref,
             hcomm_ref, ecr_ref, ecl_ref,
             hsend, hrecv, esend_r, erecv_r, esend_l, erecv_l):
        me = lax.axis_index("i")
        right = (me + 1) % N_DEV
        left = (me + 3) % N_DEV

        with jax.named_scope("barrier"):
            barrier_sem = pltpu.get_barrier_semaphore()
            for nbr in (left, right):
                pl.semaphore_signal(
                    barrier_sem, inc=1,
                    device_id=(nbr,), device_id_type=pl.DeviceIdType.MESH,
                )
            pl.semaphore_wait(barrier_sem, 2)

        def mk_rdma(h, comm_ref, send_sems, recv_sems, target):
            src_slot, dst_slot = h % 2, (h + 1) % 2
            return pltpu.make_async_remote_copy(
                src_ref=comm_ref.at[src_slot],
                dst_ref=comm_ref.at[dst_slot],
                send_sem=send_sems.at[src_slot],
                recv_sem=recv_sems.at[dst_slot],
                device_id=(target,),
                device_id_type=pl.DeviceIdType.MESH,
            )

        ridx = ridx_ref[:, :]
        e_iota = lax.broadcasted_iota(jnp.int32, (N_TOK, N_EXPERTS), 1)
        onehot = (ridx == e_iota).astype(jnp.float32)
        hist = jnp.sum(onehot, axis=0).astype(jnp.int32)
        hcomm_ref[0, 0:1, 0:N_EXPERTS] = hist.reshape(1, N_EXPERTS)
        ew_bf = ew_ref[:, :, :].astype(jnp.bfloat16)
        ecr_ref[0] = ew_bf[0:2]
        ecl_ref[0] = ew_bf[2:4]

        x_bf = x_ref[:, :].astype(jnp.bfloat16)

        def acc_expert(g, w_bf, first):
            sel = (ridx == g).astype(jnp.bfloat16)
            d = jnp.dot(x_bf * sel, w_bf, preferred_element_type=jnp.float32)
            if first:
                out_ref[:, :] = d
            else:
                out_ref[:, :] += d

        def compute_pair(h, w_r_of, w_l_of):
            owner_r = (me + (N_DEV - h)) % N_DEV
            owner_l = (me + h) % N_DEV
            for j in (0, 1):
                acc_expert(owner_r * E_LOCAL + j, w_r_of(j), h == 0 and j == 0)
            for j in (2, 3):
                acc_expert(owner_l * E_LOCAL + j, w_l_of(j - 2), False)

        prev = jnp.zeros((N_EXPERTS,), jnp.float32)
        my_rank = None
        for h in range(N_DEV - 1):
            rdma_r = mk_rdma(h, ecr_ref, esend_r, erecv_r, right)
            rdma_l = mk_rdma(h, ecl_ref, esend_l, erecv_l, left)
            rdma_h = mk_rdma(h, hcomm_ref, hsend, hrecv, right)
            with jax.named_scope(f"ering_send#hop={h}"):
                rdma_r.start()
                rdma_l.start()
                rdma_h.start()
            with jax.named_scope(f"ering_compute#hop={h}"):
                if h == 0:
                    lmat = (lax.broadcasted_iota(jnp.int32, (N_TOK, N_TOK), 0)
                            > lax.broadcasted_iota(jnp.int32, (N_TOK, N_TOK), 1)
                            ).astype(jnp.float32)
                    ranks = jnp.dot(lmat, onehot,
                                    preferred_element_type=jnp.float32)
                    my_rank = jnp.sum(ranks * onehot, axis=1, keepdims=True)
                    compute_pair(0, lambda j: ew_bf[j], lambda j: ew_bf[2 + j])
                else:
                    s = h % 2
                    compute_pair(h,
                                 lambda j, s=s: ecr_ref[s, j],
                                 lambda j, s=s: ecl_ref[s, j])
            with jax.named_scope(f"ering_wait#hop={h}"):
                rdma_r.wait()
                rdma_l.wait()
                rdma_h.wait()
            origin = (me + (N_DEV - 1 - h)) % N_DEV
            w = (origin < me).astype(jnp.float32)
            prev = prev + w * hcomm_ref[(h + 1) % 2, 0, 0:N_EXPERTS].astype(
                jnp.float32)
        with jax.named_scope("ering_compute#hop=3"):
            compute_pair(N_DEV - 1,
                         lambda j: ecr_ref[1, j],
                         lambda j: ecl_ref[1, j])

        with jax.named_scope("apply_accept"):
            my_prev = jnp.sum(onehot * prev[None, :], axis=1, keepdims=True)
            accepted = ((my_prev + my_rank) < float(CAPACITY)).astype(
                jnp.float32)
            out_ref[:, :] *= accepted

    out_shape = jax.ShapeDtypeStruct((N_TOK, D_HIDDEN), jnp.float32)
    return pl.pallas_call(
        body,
        out_shape=out_shape,
        in_specs=[pl.BlockSpec(memory_space=pltpu.VMEM)] * 3,
        out_specs=pl.BlockSpec(memory_space=pltpu.VMEM),
        scratch_shapes=[
            pltpu.VMEM((2, 8, 128), jnp.int32),
            pltpu.VMEM((2, 2, D_MODEL, D_HIDDEN), jnp.bfloat16),
            pltpu.VMEM((2, 2, D_MODEL, D_HIDDEN), jnp.bfloat16),
            pltpu.SemaphoreType.DMA((2,)),
            pltpu.SemaphoreType.DMA((2,)),
            pltpu.SemaphoreType.DMA((2,)),
            pltpu.SemaphoreType.DMA((2,)),
            pltpu.SemaphoreType.DMA((2,)),
            pltpu.SemaphoreType.DMA((2,)),
        ],
        compiler_params=pltpu.CompilerParams(collective_id=0),
    )(x, route_idx, expert_W)
